# baseline (speedup 1.0000x reference)
import functools

import jax
import jax.numpy as jnp
from jax import lax
from jax.experimental import pallas as pl
from jax.experimental.pallas import tpu as pltpu
from jax.experimental.pallas import tpu_sc as plsc

N = 10000
E = 320000
D = 128
OUT = 128
NUM_GRAPHS = 64

NC = 2
NS = 16
NW = NC * NS
CHUNK = 125
E_PAD = E
ROWS_PER_W = E_PAD // CHUNK // NW
NBLK = 2
RPB = ROWS_PER_W // NBLK
N_PAD = 10240
NODES_PER_S = N_PAD // NS


def _seg_sum_body(h_hbm, src_hbm, dst_hbm, out_hbm,
                  src_v, dst_v, rows_a, rows_b, agg_s,
                  sem_a, sem_b, ssem_a, ssem_b):
    c = lax.axis_index("c")
    s = lax.axis_index("s")
    wid = c * NS + s

    zvec = jnp.zeros((16,), jnp.float32)
    def _zrow(i, carry):
        for j in range(D // 16):
            rows_a[i, pl.ds(j * 16, 16)] = zvec
        return carry
    lax.fori_loop(0, CHUNK, _zrow, 0)
    base = s * NODES_PER_S
    ZROWS = 80
    for k in range(NODES_PER_S // ZROWS):
        pltpu.sync_copy(rows_a.at[pl.ds(0, ZROWS)],
                        agg_s.at[pl.ds(base + k * ZROWS, ZROWS)])
    plsc.subcore_barrier()

    for b in range(NBLK):
        pltpu.sync_copy(src_hbm.at[wid, b], src_v)
        pltpu.sync_copy(dst_hbm.at[wid, b], dst_v)
        pltpu.async_copy(h_hbm.at[src_v.at[0]], rows_a, sem_a)
        pltpu.async_copy(h_hbm.at[src_v.at[1]], rows_b, sem_b)
        def _pair(g, carry):
            j = 2 * g
            pltpu.make_async_copy(h_hbm.at[src_v.at[j]], rows_a, sem_a).wait()
            pltpu.async_copy(rows_a, agg_s.at[dst_v.at[j]], ssem_a, add=True)
            pltpu.make_async_copy(h_hbm.at[src_v.at[j + 1]], rows_b, sem_b).wait()
            pltpu.async_copy(rows_b, agg_s.at[dst_v.at[j + 1]], ssem_b, add=True)
            pltpu.make_async_copy(rows_a, agg_s.at[dst_v.at[j]], ssem_a).wait()
            @pl.when(j + 2 < RPB)
            def _():
                pltpu.async_copy(h_hbm.at[src_v.at[j + 2]], rows_a, sem_a)
            pltpu.make_async_copy(rows_b, agg_s.at[dst_v.at[j + 1]], ssem_b).wait()
            @pl.when(j + 3 < RPB)
            def _():
                pltpu.async_copy(h_hbm.at[src_v.at[j + 3]], rows_b, sem_b)
            return carry
        lax.fori_loop(0, RPB // 2, _pair, 0)
    plsc.subcore_barrier()

    pltpu.sync_copy(agg_s.at[pl.ds(s * NODES_PER_S, NODES_PER_S)],
                    out_hbm.at[c, pl.ds(s * NODES_PER_S, NODES_PER_S)])


_seg_sum = pl.kernel(
    _seg_sum_body,
    out_type=jax.ShapeDtypeStruct((NC, N_PAD, D), jnp.float32),
    mesh=plsc.VectorSubcoreMesh(core_axis_name="c", subcore_axis_name="s"),
    scratch_types=[
        pltpu.VMEM((RPB, CHUNK), jnp.int32),
        pltpu.VMEM((RPB, CHUNK), jnp.int32),
        pltpu.VMEM((CHUNK, D), jnp.float32),
        pltpu.VMEM((CHUNK, D), jnp.float32),
        pltpu.VMEM_SHARED((N_PAD, D), jnp.float32),
        pltpu.SemaphoreType.DMA,
        pltpu.SemaphoreType.DMA,
        pltpu.SemaphoreType.DMA,
        pltpu.SemaphoreType.DMA,
    ],
)


BLK = 1000
NBLOCKS = N // BLK


def _mlp_body(h_ref, p0_ref, p1_ref, w1_ref, b1_ref, w2_ref, b2_ref, o_ref):
    m = h_ref[...] + p0_ref[0] + p1_ref[0]
    z = jnp.maximum(
        jnp.dot(m, w1_ref[...], preferred_element_type=jnp.float32)
        + b1_ref[...], 0.0)
    o_ref[...] = jnp.maximum(
        jnp.dot(z, w2_ref[...], preferred_element_type=jnp.float32)
        + b2_ref[...], 0.0)


_ROW = pl.BlockSpec((BLK, D), lambda i: (i, 0))
_P0 = pl.BlockSpec((1, BLK, D), lambda i: (0, i, 0))
_P1 = pl.BlockSpec((1, BLK, D), lambda i: (1, i, 0))
_FULL = lambda shape: pl.BlockSpec(shape, lambda i: (0,) * len(shape))


def _mlp(h, parts, w1, b1, w2, b2):
    return pl.pallas_call(
        _mlp_body,
        grid=(NBLOCKS,),
        in_specs=[_ROW, _P0, _P1,
                  _FULL((D, D)), _FULL((1, D)), _FULL((D, D)), _FULL((1, D))],
        out_specs=_ROW,
        out_shape=jax.ShapeDtypeStruct((N, D), jnp.float32),
    )(h, parts, parts, w1, b1.reshape(1, D), w2, b2.reshape(1, D))


def _mlp_pool_body(h_ref, p0_ref, p1_ref, w1_ref, b1_ref, w2_ref, b2_ref,
                   batch_ref, wout_ref, bout_ref, out_ref, emb_ref):
    i = pl.program_id(0)
    m = h_ref[...] + p0_ref[0] + p1_ref[0]
    z = jnp.maximum(
        jnp.dot(m, w1_ref[...], preferred_element_type=jnp.float32)
        + b1_ref[...], 0.0)
    y = (jnp.dot(z, w2_ref[...], preferred_element_type=jnp.float32)
         + b2_ref[...])
    oh = (batch_ref[...]
          == lax.broadcasted_iota(jnp.int32, (1, NUM_GRAPHS), 1)
          ).astype(jnp.float32)
    part = lax.dot_general(oh, y, (((0,), (0,)), ((), ())),
                           preferred_element_type=jnp.float32)

    @pl.when(i == 0)
    def _():
        emb_ref[...] = part

    @pl.when(i > 0)
    def _():
        emb_ref[...] += part

    @pl.when(i == NBLOCKS - 1)
    def _():
        out_ref[...] = (jnp.dot(emb_ref[...], wout_ref[...],
                                preferred_element_type=jnp.float32)
                        + bout_ref[...])


def _mlp_pool(h, parts, w1, b1, w2, b2, batch2d, w_out, b_out):
    return pl.pallas_call(
        _mlp_pool_body,
        grid=(NBLOCKS,),
        in_specs=[_ROW, _P0, _P1,
                  _FULL((D, D)), _FULL((1, D)), _FULL((D, D)), _FULL((1, D)),
                  pl.BlockSpec((BLK, 1), lambda i: (i, 0)),
                  _FULL((D, OUT)), _FULL((1, OUT))],
        out_specs=[_FULL((NUM_GRAPHS, OUT)), _FULL((NUM_GRAPHS, D))],
        out_shape=[jax.ShapeDtypeStruct((NUM_GRAPHS, OUT), jnp.float32),
                   jax.ShapeDtypeStruct((NUM_GRAPHS, D), jnp.float32)],
    )(h, parts, parts, w1, b1.reshape(1, D), w2, b2.reshape(1, D),
      batch2d, w_out, b_out.reshape(1, OUT))


def kernel(x, edge_index, batch, W1_0, b1_0, W2_0, b2_0, W1_1, b1_1, W2_1,
           b2_1, W1_2, b1_2, W2_2, b2_2, W_out, b_out):
    src = edge_index[0].reshape(NW, NBLK, RPB, CHUNK)
    dst = edge_index[1].reshape(NW, NBLK, RPB, CHUNK)
    batch2d = batch.reshape(N, 1)
    h = x
    for W1, b1, W2, b2 in [(W1_0, b1_0, W2_0, b2_0), (W1_1, b1_1, W2_1, b2_1)]:
        parts = _seg_sum(h, src, dst)
        h = _mlp(h, parts, W1, b1, W2, b2)
    parts = _seg_sum(h, src, dst)
    out, emb = _mlp_pool(h, parts, W1_2, b1_2, W2_2, b2_2,
                         batch2d, W_out, b_out)
    return (out, emb)


# speedup vs baseline: 1.3129x; 1.3129x over previous
import functools

import jax
import jax.numpy as jnp
from jax import lax
from jax.experimental import pallas as pl
from jax.experimental.pallas import tpu as pltpu
from jax.experimental.pallas import tpu_sc as plsc

N = 10000
E = 320000
D = 128
OUT = 128
NUM_GRAPHS = 64

NC = 2
NS = 16
NW = NC * NS
CHUNK = 100
E_PAD = E
ROWS_PER_W = E_PAD // CHUNK // NW
NBLK = 4
RPB = ROWS_PER_W // NBLK
N_PAD = 10240
NODES_PER_S = N_PAD // NS


def _seg_sum_body(h_hbm, src_hbm, dst_hbm, out_hbm,
                  src_v, dst_v, rows_a, rows_b, rows_c, agg_s,
                  sem_a, sem_b, sem_c):
    c = lax.axis_index("c")
    s = lax.axis_index("s")
    wid = c * NS + s

    zvec = jnp.zeros((16,), jnp.float32)
    def _zrow(i, carry):
        for j in range(D // 16):
            rows_a[i, pl.ds(j * 16, 16)] = zvec
        return carry
    lax.fori_loop(0, CHUNK, _zrow, 0)
    base = s * NODES_PER_S
    ZROWS = 80
    for k in range(NODES_PER_S // ZROWS):
        pltpu.sync_copy(rows_a.at[pl.ds(0, ZROWS)],
                        agg_s.at[pl.ds(base + k * ZROWS, ZROWS)])
    plsc.subcore_barrier()

    for b in range(NBLK):
        pltpu.sync_copy(src_hbm.at[wid, b], src_v)
        pltpu.sync_copy(dst_hbm.at[wid, b], dst_v)
        pltpu.async_copy(h_hbm.at[src_v.at[0]], rows_a, sem_a)
        pltpu.async_copy(h_hbm.at[src_v.at[1]], rows_b, sem_b)
        pltpu.async_copy(h_hbm.at[src_v.at[2]], rows_c, sem_c)
        def _trip(g, carry):
            j = 3 * g
            for rv, sm, o in ((rows_a, sem_a, 0), (rows_b, sem_b, 1),
                              (rows_c, sem_c, 2)):
                pltpu.make_async_copy(h_hbm.at[src_v.at[j + o]], rv, sm).wait()
                pltpu.sync_copy(rv, agg_s.at[dst_v.at[j + o]], add=True)
                @pl.when(j + o + 3 < RPB)
                def _(rv=rv, sm=sm, o=o):
                    pltpu.async_copy(h_hbm.at[src_v.at[j + o + 3]], rv, sm)
            return carry
        lax.fori_loop(0, RPB // 3, _trip, 0)
        jt = RPB - 1
        pltpu.make_async_copy(h_hbm.at[src_v.at[jt]], rows_a, sem_a).wait()
        pltpu.sync_copy(rows_a, agg_s.at[dst_v.at[jt]], add=True)
    plsc.subcore_barrier()

    pltpu.sync_copy(agg_s.at[pl.ds(s * NODES_PER_S, NODES_PER_S)],
                    out_hbm.at[c, pl.ds(s * NODES_PER_S, NODES_PER_S)])


_seg_sum = pl.kernel(
    _seg_sum_body,
    out_type=jax.ShapeDtypeStruct((NC, N_PAD, D), jnp.float32),
    mesh=plsc.VectorSubcoreMesh(core_axis_name="c", subcore_axis_name="s"),
    scratch_types=[
        pltpu.VMEM((RPB, CHUNK), jnp.int32),
        pltpu.VMEM((RPB, CHUNK), jnp.int32),
        pltpu.VMEM((CHUNK, D), jnp.float32),
        pltpu.VMEM((CHUNK, D), jnp.float32),
        pltpu.VMEM((CHUNK, D), jnp.float32),
        pltpu.VMEM_SHARED((N_PAD, D), jnp.float32),
        pltpu.SemaphoreType.DMA,
        pltpu.SemaphoreType.DMA,
        pltpu.SemaphoreType.DMA,
    ],
)


BLK = 1000
NBLOCKS = N // BLK


def _mlp_body(h_ref, p0_ref, p1_ref, w1_ref, b1_ref, w2_ref, b2_ref, o_ref):
    m = h_ref[...] + p0_ref[0] + p1_ref[0]
    z = jnp.maximum(
        jnp.dot(m, w1_ref[...], preferred_element_type=jnp.float32)
        + b1_ref[...], 0.0)
    o_ref[...] = jnp.maximum(
        jnp.dot(z, w2_ref[...], preferred_element_type=jnp.float32)
        + b2_ref[...], 0.0)


_ROW = pl.BlockSpec((BLK, D), lambda i: (i, 0))
_P0 = pl.BlockSpec((1, BLK, D), lambda i: (0, i, 0))
_P1 = pl.BlockSpec((1, BLK, D), lambda i: (1, i, 0))
_FULL = lambda shape: pl.BlockSpec(shape, lambda i: (0,) * len(shape))


def _mlp(h, parts, w1, b1, w2, b2):
    return pl.pallas_call(
        _mlp_body,
        grid=(NBLOCKS,),
        in_specs=[_ROW, _P0, _P1,
                  _FULL((D, D)), _FULL((1, D)), _FULL((D, D)), _FULL((1, D))],
        out_specs=_ROW,
        out_shape=jax.ShapeDtypeStruct((N, D), jnp.float32),
    )(h, parts, parts, w1, b1.reshape(1, D), w2, b2.reshape(1, D))


def _mlp_pool_body(h_ref, p0_ref, p1_ref, w1_ref, b1_ref, w2_ref, b2_ref,
                   batch_ref, wout_ref, bout_ref, out_ref, emb_ref):
    i = pl.program_id(0)
    m = h_ref[...] + p0_ref[0] + p1_ref[0]
    z = jnp.maximum(
        jnp.dot(m, w1_ref[...], preferred_element_type=jnp.float32)
        + b1_ref[...], 0.0)
    y = (jnp.dot(z, w2_ref[...], preferred_element_type=jnp.float32)
         + b2_ref[...])
    oh = (batch_ref[...]
          == lax.broadcasted_iota(jnp.int32, (1, NUM_GRAPHS), 1)
          ).astype(jnp.float32)
    part = lax.dot_general(oh, y, (((0,), (0,)), ((), ())),
                           preferred_element_type=jnp.float32)

    @pl.when(i == 0)
    def _():
        emb_ref[...] = part

    @pl.when(i > 0)
    def _():
        emb_ref[...] += part

    @pl.when(i == NBLOCKS - 1)
    def _():
        out_ref[...] = (jnp.dot(emb_ref[...], wout_ref[...],
                                preferred_element_type=jnp.float32)
                        + bout_ref[...])


def _mlp_pool(h, parts, w1, b1, w2, b2, batch2d, w_out, b_out):
    return pl.pallas_call(
        _mlp_pool_body,
        grid=(NBLOCKS,),
        in_specs=[_ROW, _P0, _P1,
                  _FULL((D, D)), _FULL((1, D)), _FULL((D, D)), _FULL((1, D)),
                  pl.BlockSpec((BLK, 1), lambda i: (i, 0)),
                  _FULL((D, OUT)), _FULL((1, OUT))],
        out_specs=[_FULL((NUM_GRAPHS, OUT)), _FULL((NUM_GRAPHS, D))],
        out_shape=[jax.ShapeDtypeStruct((NUM_GRAPHS, OUT), jnp.float32),
                   jax.ShapeDtypeStruct((NUM_GRAPHS, D), jnp.float32)],
    )(h, parts, parts, w1, b1.reshape(1, D), w2, b2.reshape(1, D),
      batch2d, w_out, b_out.reshape(1, OUT))


def kernel(x, edge_index, batch, W1_0, b1_0, W2_0, b2_0, W1_1, b1_1, W2_1,
           b2_1, W1_2, b1_2, W2_2, b2_2, W_out, b_out):
    src = edge_index[0].reshape(NW, NBLK, RPB, CHUNK)
    dst = edge_index[1].reshape(NW, NBLK, RPB, CHUNK)
    batch2d = batch.reshape(N, 1)
    h = x
    for W1, b1, W2, b2 in [(W1_0, b1_0, W2_0, b2_0), (W1_1, b1_1, W2_1, b2_1)]:
        parts = _seg_sum(h, src, dst)
        h = _mlp(h, parts, W1, b1, W2, b2)
    parts = _seg_sum(h, src, dst)
    out, emb = _mlp_pool(h, parts, W1_2, b1_2, W2_2, b2_2,
                         batch2d, W_out, b_out)
    return (out, emb)
